# baseline (speedup 1.0000x reference)
import functools

import jax
import jax.numpy as jnp
from jax import lax
from jax.experimental import pallas as pl
from jax.experimental.pallas import tpu as pltpu
from jax.experimental.pallas import tpu_sc as plsc

_NC = 2
_NS = 16
_L = 16


def _make_sc_gather(n_values, n_rows, n_cols, chunk_rows):
  nw = _NC * _NS
  rows_per_worker = n_rows // nw
  nchunks = rows_per_worker // chunk_rows
  n_full = n_cols // _L
  col_starts = [v * _L for v in range(n_full)]
  if n_full * _L < n_cols:
    col_starts.append(n_cols - _L)
  mesh = plsc.VectorSubcoreMesh(
      core_axis_name="c", subcore_axis_name="s",
      num_cores=_NC, num_subcores=_NS)

  @functools.partial(
      pl.kernel,
      out_type=jax.ShapeDtypeStruct((n_rows, n_cols), jnp.float32),
      mesh=mesh,
      scratch_types=[
          pltpu.VMEM((n_values,), jnp.float32),
          pltpu.VMEM((2, chunk_rows, n_cols), jnp.int32),
          pltpu.VMEM((2, chunk_rows, n_cols), jnp.float32),
          pltpu.SemaphoreType.DMA,
          pltpu.SemaphoreType.DMA,
          pltpu.SemaphoreType.DMA,
          pltpu.SemaphoreType.DMA,
      ],
      compiler_params=pltpu.CompilerParams(needs_layout_passes=False),
  )
  def gather_kernel(vals_hbm, idx_hbm, out_hbm, vals_v, idx_v, out_v,
                    sem_in0, sem_in1, sem_out0, sem_out1):
    sems_in = (sem_in0, sem_in1)
    sems_out = (sem_out0, sem_out1)
    wid = lax.axis_index("s") * _NC + lax.axis_index("c")
    base = wid * rows_per_worker

    def start_in(ci):
      r0 = base + ci * chunk_rows
      return pltpu.async_copy(
          idx_hbm.at[pl.ds(r0, chunk_rows), :], idx_v.at[ci % 2],
          sems_in[ci % 2])

    in_copies = [None] * nchunks
    out_copies = [None] * nchunks
    in_copies[0] = start_in(0)
    pltpu.sync_copy(vals_hbm, vals_v)
    for ci in range(nchunks):
      buf = ci % 2
      if ci + 1 < nchunks:
        in_copies[ci + 1] = start_in(ci + 1)
      in_copies[ci].wait()
      if ci >= 2:
        out_copies[ci - 2].wait()

      @plsc.parallel_loop(0, chunk_rows, step=1, unroll=4)
      def body(r, buf=buf):
        for c in col_starts:
          out_v[buf, r, pl.ds(c, _L)] = plsc.load_gather(
              vals_v, [idx_v[buf, r, pl.ds(c, _L)]])

      r0 = base + ci * chunk_rows
      out_copies[ci] = pltpu.async_copy(
          out_v.at[buf], out_hbm.at[pl.ds(r0, chunk_rows), :], sems_out[buf])

    out_copies[nchunks - 2].wait()
    out_copies[nchunks - 1].wait()

  return gather_kernel


def kernel(values, index):
  n_rows, n_cols = index.shape
  return _make_sc_gather(values.shape[0], n_rows, n_cols, 64)(values, index)


# speedup vs baseline: 1.0035x; 1.0035x over previous
import functools

import jax
import jax.numpy as jnp
from jax import lax
from jax.experimental import pallas as pl
from jax.experimental.pallas import tpu as pltpu
from jax.experimental.pallas import tpu_sc as plsc

_NC = 2
_NS = 16
_L = 16


def _make_sc_gather(n_values, n_rows, n_cols, chunk_rows):
  nw = _NC * _NS
  rows_per_worker = n_rows // nw
  nchunks = rows_per_worker // chunk_rows
  n_full = n_cols // _L
  col_starts = [v * _L for v in range(n_full)]
  if n_full * _L < n_cols:
    col_starts.append(n_cols - _L)
  mesh = plsc.VectorSubcoreMesh(
      core_axis_name="c", subcore_axis_name="s",
      num_cores=_NC, num_subcores=_NS)

  @functools.partial(
      pl.kernel,
      out_type=jax.ShapeDtypeStruct((n_rows, n_cols), jnp.float32),
      mesh=mesh,
      scratch_types=[
          pltpu.VMEM((n_values,), jnp.float32),
          pltpu.VMEM((2, chunk_rows, n_cols), jnp.int32),
          pltpu.VMEM((2, chunk_rows, n_cols), jnp.float32),
          pltpu.SemaphoreType.DMA,
          pltpu.SemaphoreType.DMA,
          pltpu.SemaphoreType.DMA,
          pltpu.SemaphoreType.DMA,
      ],
      compiler_params=pltpu.CompilerParams(
          needs_layout_passes=False, skip_device_barrier=True),
  )
  def gather_kernel(vals_hbm, idx_hbm, out_hbm, vals_v, idx_v, out_v,
                    sem_in0, sem_in1, sem_out0, sem_out1):
    sems_in = (sem_in0, sem_in1)
    sems_out = (sem_out0, sem_out1)
    wid = lax.axis_index("s") * _NC + lax.axis_index("c")
    base = wid * rows_per_worker

    def start_in(ci):
      r0 = base + ci * chunk_rows
      return pltpu.async_copy(
          idx_hbm.at[pl.ds(r0, chunk_rows), :], idx_v.at[ci % 2],
          sems_in[ci % 2])

    in_copies = [None] * nchunks
    out_copies = [None] * nchunks
    in_copies[0] = start_in(0)
    pltpu.sync_copy(vals_hbm, vals_v)
    for ci in range(nchunks):
      buf = ci % 2
      if ci + 1 < nchunks:
        in_copies[ci + 1] = start_in(ci + 1)
      in_copies[ci].wait()
      if ci >= 2:
        out_copies[ci - 2].wait()

      @plsc.parallel_loop(0, chunk_rows, step=1, unroll=4)
      def body(r, buf=buf):
        for c in col_starts:
          out_v[buf, r, pl.ds(c, _L)] = plsc.load_gather(
              vals_v, [idx_v[buf, r, pl.ds(c, _L)]])

      r0 = base + ci * chunk_rows
      out_copies[ci] = pltpu.async_copy(
          out_v.at[buf], out_hbm.at[pl.ds(r0, chunk_rows), :], sems_out[buf])

    out_copies[nchunks - 2].wait()
    out_copies[nchunks - 1].wait()

  return gather_kernel


def kernel(values, index):
  n_rows, n_cols = index.shape
  return _make_sc_gather(values.shape[0], n_rows, n_cols, 64)(values, index)


# speedup vs baseline: 1.0039x; 1.0004x over previous
import functools

import jax
import jax.numpy as jnp
from jax import lax
from jax.experimental import pallas as pl
from jax.experimental.pallas import tpu as pltpu
from jax.experimental.pallas import tpu_sc as plsc

_NC = 2
_NS = 16
_L = 16


def _make_sc_gather(n_values, n_rows, n_cols, chunk_rows):
  nw = _NC * _NS
  rows_per_worker = n_rows // nw
  nchunks = rows_per_worker // chunk_rows
  n_full = n_cols // _L
  col_starts = [v * _L for v in range(n_full)]
  if n_full * _L < n_cols:
    col_starts.append(n_cols - _L)
  mesh = plsc.VectorSubcoreMesh(
      core_axis_name="c", subcore_axis_name="s",
      num_cores=_NC, num_subcores=_NS)

  @functools.partial(
      pl.kernel,
      out_type=jax.ShapeDtypeStruct((n_rows, n_cols), jnp.float32),
      mesh=mesh,
      scratch_types=[
          pltpu.VMEM((n_values,), jnp.float32),
          pltpu.VMEM((2, chunk_rows, n_cols), jnp.int32),
          pltpu.VMEM((2, chunk_rows, n_cols), jnp.float32),
          pltpu.SemaphoreType.DMA,
          pltpu.SemaphoreType.DMA,
          pltpu.SemaphoreType.DMA,
          pltpu.SemaphoreType.DMA,
      ],
      compiler_params=pltpu.CompilerParams(needs_layout_passes=False),
  )
  def gather_kernel(vals_hbm, idx_hbm, out_hbm, vals_v, idx_v, out_v,
                    sem_in0, sem_in1, sem_out0, sem_out1):
    sems_in = (sem_in0, sem_in1)
    sems_out = (sem_out0, sem_out1)
    wid = lax.axis_index("s") * _NC + lax.axis_index("c")
    base = wid * rows_per_worker

    def start_in(ci):
      r0 = base + ci * chunk_rows
      return pltpu.async_copy(
          idx_hbm.at[pl.ds(r0, chunk_rows), :], idx_v.at[ci % 2],
          sems_in[ci % 2])

    in_copies = [None] * nchunks
    out_copies = [None] * nchunks
    in_copies[0] = start_in(0)
    pltpu.sync_copy(vals_hbm, vals_v)
    for ci in range(nchunks):
      buf = ci % 2
      if ci + 1 < nchunks:
        in_copies[ci + 1] = start_in(ci + 1)
      in_copies[ci].wait()
      if ci >= 2:
        out_copies[ci - 2].wait()

      @plsc.parallel_loop(0, chunk_rows, step=1, unroll=4)
      def body(r, buf=buf):
        for c in col_starts:
          out_v[buf, r, pl.ds(c, _L)] = plsc.load_gather(
              vals_v, [idx_v[buf, r, pl.ds(c, _L)]])

      r0 = base + ci * chunk_rows
      out_copies[ci] = pltpu.async_copy(
          out_v.at[buf], out_hbm.at[pl.ds(r0, chunk_rows), :], sems_out[buf])

    out_copies[nchunks - 2].wait()
    out_copies[nchunks - 1].wait()

  return gather_kernel


def kernel(values, index):
  n_rows, n_cols = index.shape
  return _make_sc_gather(values.shape[0], n_rows, n_cols, 64)(values, index)
